# baseline (speedup 1.0000x reference)
import functools

import jax
import jax.numpy as jnp
from jax import lax
from jax.experimental import pallas as pl
from jax.experimental.pallas import tpu as pltpu
from jax.experimental.pallas import tpu_sc as plsc

_LANES = 16
_C = 1600
_G = 128
_LIST = 1728


@functools.lru_cache(maxsize=None)
def _make_segmax(n_nodes, n_edges, d):
    info = plsc.get_sparse_core_info()
    nw = info.num_cores * info.num_subcores
    rpw = ((n_nodes + nw * 8 - 1) // (nw * 8)) * 8
    npad = rpw * nw
    nchunk = n_edges // _C
    mesh = plsc.VectorSubcoreMesh(core_axis_name="c", subcore_axis_name="s")

    @functools.partial(
        pl.kernel,
        out_type=jax.ShapeDtypeStruct((npad, d), jnp.float32),
        mesh=mesh,
        compiler_params=pltpu.CompilerParams(needs_layout_passes=False),
        scratch_types=[
            pltpu.VMEM((rpw, d), jnp.float32),
            pltpu.VMEM((_G, d), jnp.float32),
            pltpu.VMEM((_C,), jnp.int32),
            pltpu.VMEM((_C,), jnp.int32),
            pltpu.VMEM((_LIST,), jnp.int32),
            pltpu.VMEM((_LIST,), jnp.int32),
            pltpu.SemaphoreType.DMA,
        ],
    )
    def segmax(h_hbm, src_hbm, dst_hbm, out_hbm,
               agg, rows, srcc, dstc, lsrc, lld, sem):
        wid = lax.axis_index("s") * info.num_cores + lax.axis_index("c")
        base = wid * rpw
        neg = jnp.full((_LANES,), -jnp.inf, jnp.float32)

        def init_body(r, carry):
            for j in range(d // _LANES):
                agg[r, pl.ds(j * _LANES, _LANES)] = neg
            return carry
        lax.fori_loop(0, rpw, init_body, 0)

        zero = jnp.zeros((_LANES,), jnp.int32)

        def zero_body(i, carry):
            lsrc[pl.ds(i * _LANES, _LANES)] = zero
            return carry
        lax.fori_loop(0, _LIST // _LANES, zero_body, 0)

        def chunk_body(c, carry):
            pltpu.sync_copy(src_hbm.at[pl.ds(c * _C, _C)], srcc)
            pltpu.sync_copy(dst_hbm.at[pl.ds(c * _C, _C)], dstc)

            def compact_body(i, cnt):
                d16 = dstc[pl.ds(i * _LANES, _LANES)]
                s16 = srcc[pl.ds(i * _LANES, _LANES)]
                m = (d16 >= base) & (d16 < base + rpw)
                pref = plsc.cumsum(m.astype(jnp.int32))
                idx = cnt + pref - 1
                plsc.store_scatter(lsrc, [idx], s16, mask=m)
                plsc.store_scatter(lld, [idx], d16 - base, mask=m)
                return cnt + jnp.max(pref)

            cnt = lax.fori_loop(0, _C // _LANES, compact_body, jnp.int32(0))

            def batch_body(b, carry):
                lo = b * _G
                pltpu.async_copy(h_hbm.at[lsrc.at[pl.ds(lo, _G)]], rows,
                                 sem).wait()
                hi = jnp.minimum(cnt, lo + _G)

                def edge_body(e, carry2):
                    ld = lld[pl.ds(e, _LANES)][0]
                    eb = e - lo
                    for j in range(d // _LANES):
                        sl = pl.ds(j * _LANES, _LANES)
                        agg[ld, sl] = jnp.maximum(agg[ld, sl], rows[eb, sl])
                    return carry2
                lax.fori_loop(lo, hi, edge_body, 0)
                return carry

            nb = (cnt + _G - 1) >> 7
            lax.fori_loop(0, nb, batch_body, 0)
            return carry

        lax.fori_loop(0, nchunk, chunk_body, 0)

        def fin_body(r, carry):
            for j in range(d // _LANES):
                sl = pl.ds(j * _LANES, _LANES)
                v = agg[r, sl]
                agg[r, sl] = jnp.where(v == -jnp.inf, 0.0, v)
            return carry
        lax.fori_loop(0, rpw, fin_body, 0)
        pltpu.sync_copy(agg, out_hbm.at[pl.ds(base, rpw)])

    return segmax, npad


def _segment_max(h, src, dst):
    n, d = h.shape
    fn, npad = _make_segmax(n, src.shape[0], d)
    return fn(h, src, dst)[:n]


def _ln(t, g, beta):
    m = jnp.mean(t, axis=-1, keepdims=True)
    v = jnp.mean((t - m) ** 2, axis=-1, keepdims=True)
    return (t - m) / jnp.sqrt(v + 1e-5) * g + beta


def _dense0_body(x_ref, w1_ref, b1_ref, g_ref, bt_ref, w2_ref, b2_ref, o_ref):
    t = jnp.dot(x_ref[...], w1_ref[...],
                preferred_element_type=jnp.float32) + b1_ref[...]
    t = jnp.maximum(_ln(t, g_ref[...], bt_ref[...]), 0.0)
    o_ref[...] = jnp.dot(t, w2_ref[...],
                         preferred_element_type=jnp.float32) + b2_ref[...]


def _dense1_body(a_ref, b_ref, w1a_ref, w1b_ref, b1_ref, g_ref, bt_ref,
                 w2_ref, b2_ref, o_ref):
    t = (jnp.dot(a_ref[...], w1a_ref[...], preferred_element_type=jnp.float32)
         + jnp.dot(b_ref[...], w1b_ref[...],
                   preferred_element_type=jnp.float32) + b1_ref[...])
    t = jnp.maximum(_ln(t, g_ref[...], bt_ref[...]), 0.0)
    o_ref[...] = jnp.dot(t, w2_ref[...],
                         preferred_element_type=jnp.float32) + b2_ref[...]


def _full(shape):
    return pl.BlockSpec(shape, lambda *i: tuple(0 for _ in shape))


def _dense0(x, w1, b1, g, beta, w2, b2):
    n, din = x.shape
    hid = w1.shape[1]
    dout = w2.shape[1]
    bs = 1000
    return pl.pallas_call(
        _dense0_body,
        grid=(n // bs,),
        in_specs=[pl.BlockSpec((bs, din), lambda i: (i, 0)),
                  _full((din, hid)), _full((1, hid)), _full((1, hid)),
                  _full((1, hid)), _full((hid, dout)), _full((1, dout))],
        out_specs=pl.BlockSpec((bs, dout), lambda i: (i, 0)),
        out_shape=jax.ShapeDtypeStruct((n, dout), jnp.float32),
    )(x, w1, b1.reshape(1, -1), g.reshape(1, -1), beta.reshape(1, -1),
      w2, b2.reshape(1, -1))


def _dense1(a, b, w1, b1, g, beta, w2, b2):
    n, din = a.shape
    hid = w1.shape[1]
    dout = w2.shape[1]
    bs = 1000
    w1a = w1[:din]
    w1b = w1[din:]
    return pl.pallas_call(
        _dense1_body,
        grid=(n // bs,),
        in_specs=[pl.BlockSpec((bs, din), lambda i: (i, 0)),
                  pl.BlockSpec((bs, din), lambda i: (i, 0)),
                  _full((din, hid)), _full((din, hid)), _full((1, hid)),
                  _full((1, hid)), _full((1, hid)), _full((hid, dout)),
                  _full((1, dout))],
        out_specs=pl.BlockSpec((bs, dout), lambda i: (i, 0)),
        out_shape=jax.ShapeDtypeStruct((n, dout), jnp.float32),
    )(a, b, w1a, w1b, b1.reshape(1, -1), g.reshape(1, -1),
      beta.reshape(1, -1), w2, b2.reshape(1, -1))


def _pool_body(h_ref, a_ref, o_ref):
    left = jnp.max(h_ref[...], axis=1)
    right = jnp.max(a_ref[...], axis=1)
    o_ref[...] = jnp.concatenate([left, right], axis=1)


def _pool(h, a, group):
    n, d = h.shape
    nc = n // group
    hr = h.reshape(nc, group, d)
    ar = a.reshape(nc, group, d)
    bs = 200
    return pl.pallas_call(
        _pool_body,
        grid=(nc // bs,),
        in_specs=[pl.BlockSpec((bs, group, d), lambda i: (i, 0, 0)),
                  pl.BlockSpec((bs, group, d), lambda i: (i, 0, 0))],
        out_specs=pl.BlockSpec((bs, 2 * d), lambda i: (i, 0)),
        out_shape=jax.ShapeDtypeStruct((nc, 2 * d), jnp.float32),
    )(hr, ar)


def _tail_body(p_ref, vl_ref, qw_ref, qb_ref, kw_ref, kb_ref, vw_ref, vb_ref,
               pw1_ref, pb1_ref, pg_ref, pbt_ref, pw2_ref, pb2_ref, o_ref,
               *, batch, t):
    pooled = p_ref[...]
    nc = pooled.shape[0]
    ssq = jnp.sum(pooled * pooled, axis=0, keepdims=True)
    pn = pooled / jnp.sqrt(ssq)
    q = jnp.dot(pn, qw_ref[...], preferred_element_type=jnp.float32) + qb_ref[...]
    k = jnp.dot(pn, kw_ref[...], preferred_element_type=jnp.float32) + kb_ref[...]
    v = jnp.dot(pn, vw_ref[...], preferred_element_type=jnp.float32) + vb_ref[...]
    cols = lax.broadcasted_iota(jnp.int32, (batch, nc), 1)
    bidx = lax.broadcasted_iota(jnp.int32, (batch, nc), 0)
    sel = (cols == bidx * t).astype(jnp.float32)
    q0 = jnp.dot(sel, q, preferred_element_type=jnp.float32)
    scores = lax.dot_general(q0, k, (((1,), (1,)), ((), ())),
                             preferred_element_type=jnp.float32)
    seg = (cols // t) == bidx
    s_in = cols % t
    keep = s_in <= vl_ref[...]
    sc = jnp.where(seg & keep, scores,
                   jnp.where(seg, jnp.float32(-1e-6), -jnp.inf))
    mx = jnp.max(sc, axis=-1, keepdims=True)
    e = jnp.exp(sc - mx)
    attn = e / jnp.sum(e, axis=-1, keepdims=True)
    h0 = jnp.dot(attn, v, preferred_element_type=jnp.float32)
    p = jnp.dot(h0, pw1_ref[...], preferred_element_type=jnp.float32) + pb1_ref[...]
    p = jnp.maximum(_ln(p, pg_ref[...], pbt_ref[...]), 0.0)
    o_ref[...] = jnp.dot(p, pw2_ref[...],
                         preferred_element_type=jnp.float32) + pb2_ref[...]


def _tail(pooled, valid_len, t, q_w, q_b, k_w, k_b, v_w, v_b,
          pred_w1, pred_b1, pred_g, pred_beta, pred_w2, pred_b2):
    nc, p = pooled.shape
    batch = nc // t
    ggw = q_w.shape[1]
    out = pred_w2.shape[1]
    pw = pred_w1.shape[1]
    return pl.pallas_call(
        functools.partial(_tail_body, batch=batch, t=t),
        in_specs=[_full((nc, p)), _full((batch, 1)),
                  _full((p, ggw)), _full((1, ggw)),
                  _full((p, ggw)), _full((1, ggw)),
                  _full((p, ggw)), _full((1, ggw)),
                  _full((ggw, pw)), _full((1, pw)), _full((1, pw)),
                  _full((1, pw)), _full((pw, out)), _full((1, out))],
        out_specs=_full((batch, out)),
        out_shape=jax.ShapeDtypeStruct((batch, out), jnp.float32),
    )(pooled, valid_len.reshape(batch, 1),
      q_w, q_b.reshape(1, -1), k_w, k_b.reshape(1, -1),
      v_w, v_b.reshape(1, -1), pred_w1, pred_b1.reshape(1, -1),
      pred_g.reshape(1, -1), pred_beta.reshape(1, -1),
      pred_w2, pred_b2.reshape(1, -1))


def kernel(x, edge_index, cluster, valid_len, time_step_len,
           gl0_w1, gl0_b1, gl0_g, gl0_beta, gl0_w2, gl0_b2,
           gl1_w1, gl1_b1, gl1_g, gl1_beta, gl1_w2, gl1_b2,
           q_w, q_b, k_w, k_b, v_w, v_b,
           pred_w1, pred_b1, pred_g, pred_beta, pred_w2, pred_b2):
    src = edge_index[0]
    dst = edge_index[1]
    n = x.shape[0]

    h0 = _dense0(x, gl0_w1, gl0_b1, gl0_g, gl0_beta, gl0_w2, gl0_b2)
    agg0 = _segment_max(h0, src, dst)
    h1 = _dense1(h0, agg0, gl1_w1, gl1_b1, gl1_g, gl1_beta, gl1_w2, gl1_b2)
    agg1 = _segment_max(h1, src, dst)

    t = 50
    num_clusters = valid_len.shape[0] * t
    group = n // num_clusters
    pooled = _pool(h1, agg1, group)
    return _tail(pooled, valid_len, t, q_w, q_b, k_w, k_b, v_w, v_b,
                 pred_w1, pred_b1, pred_g, pred_beta, pred_w2, pred_b2)


# speedup vs baseline: 7.7434x; 7.7434x over previous
import functools

import jax
import jax.numpy as jnp
from jax import lax
from jax.experimental import pallas as pl
from jax.experimental.pallas import tpu as pltpu
from jax.experimental.pallas import tpu_sc as plsc

_LANES = 16
_C = 3200
_G = 128
_UNROLL = 4
_LIST = 3328


@functools.lru_cache(maxsize=None)
def _make_segmax(n_nodes, n_edges, d):
    info = plsc.get_sparse_core_info()
    nw = info.num_cores * info.num_subcores
    rpw = ((n_nodes + nw * 8 - 1) // (nw * 8)) * 8
    npad = rpw * nw
    nchunk = n_edges // _C
    mesh = plsc.VectorSubcoreMesh(core_axis_name="c", subcore_axis_name="s")

    scratch = [
        pltpu.VMEM((rpw + 8, d), jnp.float32),
        pltpu.VMEM((_G, d), jnp.int32),
        pltpu.VMEM((_C,), jnp.int32),
        pltpu.VMEM((_C,), jnp.int32),
        pltpu.VMEM((_LIST,), jnp.int32),
        pltpu.VMEM((_LIST,), jnp.int32),
        pltpu.SemaphoreType.DMA,
        pltpu.SemaphoreType.DMA,
        pltpu.VMEM_SHARED((npad // 2, d), jnp.int32),
    ]

    @functools.partial(
        pl.kernel,
        out_type=jax.ShapeDtypeStruct((npad, d), jnp.float32),
        mesh=mesh,
        compiler_params=pltpu.CompilerParams(needs_layout_passes=False),
        scratch_types=scratch,
    )
    def segmax(h_hbm, src_hbm, dst_hbm, out_hbm,
               agg, rows, srcc, dstc, lsrc, lld, sem, sem2, shared):
        wid = lax.axis_index("s") * info.num_cores + lax.axis_index("c")
        base = wid * rpw
        ss = lax.axis_index("s")
        strip = npad // 2 // info.num_subcores
        pltpu.sync_copy(h_hbm.at[pl.ds(ss * strip, strip)],
                        shared.at[pl.ds(ss * strip, strip)])
        neg = jnp.full((_LANES,), -jnp.inf, jnp.float32)

        def init_body(r, carry):
            for j in range(d // _LANES):
                agg[r, pl.ds(j * _LANES, _LANES)] = neg
            return carry
        lax.fori_loop(0, rpw, init_body, 0)

        zero = jnp.zeros((_LANES,), jnp.int32)

        def zero_body(i, carry):
            lsrc[pl.ds(i * _LANES, _LANES)] = zero
            return carry
        lax.fori_loop(0, _LIST // _LANES, zero_body, 0)
        plsc.subcore_barrier()

        dummy = jnp.full((_LANES,), rpw, jnp.int32)

        def chunk_body(c, carry):
            pltpu.sync_copy(src_hbm.at[pl.ds(c * _C, _C)], srcc)
            pltpu.sync_copy(dst_hbm.at[pl.ds(c * _C, _C)], dstc)

            def compact_body(i, cnt_vec):
                subs = []
                for k in range(_UNROLL):
                    off = (i * _UNROLL + k) * _LANES
                    d16 = dstc[pl.ds(off, _LANES)]
                    s16 = srcc[pl.ds(off, _LANES)]
                    m = (d16 >= base) & (d16 < base + rpw)
                    pref = plsc.cumsum(m.astype(jnp.int32))
                    pop = plsc.all_reduce_population_count(m)
                    subs.append((d16, s16, m, pref, pop))
                acc = cnt_vec
                for d16, s16, m, pref, pop in subs:
                    idx = acc + pref - 1
                    plsc.store_scatter(lsrc, [idx], s16 >> 1, mask=m)
                    tag = (d16 - base) | ((s16 & 1) << 16)
                    plsc.store_scatter(lld, [idx], tag, mask=m)
                    acc = acc + pop
                return acc

            cnt_vec = lax.fori_loop(0, _C // (_LANES * _UNROLL), compact_body,
                                    jnp.zeros((_LANES,), jnp.int32))
            cnt = jnp.max(cnt_vec)
            pad_idx = cnt + lax.iota(jnp.int32, _LANES)
            plsc.store_scatter(lld, [pad_idx], dummy)
            ng = (cnt + 15) >> 4
            nb = (cnt + _G - 1) >> 7

            def batch_body(b, carry):
                lo = b * _G
                pltpu.async_copy(shared.at[lsrc.at[pl.ds(lo, _G)]],
                                 rows, sem).wait()
                g_hi = jnp.minimum(ng, (b + 1) * (_G // _LANES))

                def group_body(g, carry2):
                    tag_vec = lld[pl.ds(g * _LANES, _LANES)]
                    rbase = g * _LANES - lo
                    for l in range(_LANES):
                        tag = tag_vec[l]
                        ld = tag & 0xFFFF
                        half = (tag >> 16) * (d // 2)
                        rb = rbase + l
                        for j in range(d // 32):
                            packed = rows[rb, pl.ds(half + j * _LANES,
                                                    _LANES)]
                            pb = plsc.bitcast(packed, jnp.bfloat16)
                            va, vb = plsc.unpack(
                                pb, format=plsc.PackFormat.INTERLEAVED)
                            sa = pl.ds(j * 32, _LANES)
                            sb = pl.ds(j * 32 + _LANES, _LANES)
                            agg[ld, sa] = jnp.maximum(agg[ld, sa], va)
                            agg[ld, sb] = jnp.maximum(agg[ld, sb], vb)
                    return carry2
                lax.fori_loop(b * (_G // _LANES), g_hi, group_body, 0)
                return carry

            lax.fori_loop(0, nb, batch_body, 0)
            return carry

        lax.fori_loop(0, nchunk, chunk_body, 0)

        def fin_body(r, carry):
            for j in range(d // _LANES):
                sl = pl.ds(j * _LANES, _LANES)
                v = agg[r, sl]
                agg[r, sl] = jnp.where(v == -jnp.inf, 0.0, v)
            return carry
        lax.fori_loop(0, rpw, fin_body, 0)
        pltpu.sync_copy(agg.at[pl.ds(0, rpw)], out_hbm.at[pl.ds(base, rpw)])

    return segmax, npad


def _unpack_perm(d):
    perm = []
    for c in range(d):
        g, k = divmod(c, 32)
        perm.append(32 * g + (k // 2 if k % 2 == 0 else 16 + k // 2))
    return jnp.array(perm, dtype=jnp.int32)


def _segment_max(h, src, dst):
    n, d = h.shape
    fn, npad = _make_segmax(n, src.shape[0], d)
    hb = jnp.pad(h, ((0, npad - n), (0, 0))).astype(jnp.bfloat16)
    hw = jax.lax.bitcast_convert_type(
        hb.reshape(npad // 2, d, 2), jnp.int32)
    out = fn(hw, src, dst)
    return out[:n, _unpack_perm(d)]


def _ln(t, g, beta):
    m = jnp.mean(t, axis=-1, keepdims=True)
    v = jnp.mean((t - m) ** 2, axis=-1, keepdims=True)
    return (t - m) / jnp.sqrt(v + 1e-5) * g + beta


def _dense0_body(x_ref, w1_ref, b1_ref, g_ref, bt_ref, w2_ref, b2_ref, o_ref):
    t = jnp.dot(x_ref[...], w1_ref[...],
                preferred_element_type=jnp.float32) + b1_ref[...]
    t = jnp.maximum(_ln(t, g_ref[...], bt_ref[...]), 0.0)
    o_ref[...] = jnp.dot(t, w2_ref[...],
                         preferred_element_type=jnp.float32) + b2_ref[...]


def _dense1_body(a_ref, b_ref, w1a_ref, w1b_ref, b1_ref, g_ref, bt_ref,
                 w2_ref, b2_ref, o_ref):
    t = (jnp.dot(a_ref[...], w1a_ref[...], preferred_element_type=jnp.float32)
         + jnp.dot(b_ref[...], w1b_ref[...],
                   preferred_element_type=jnp.float32) + b1_ref[...])
    t = jnp.maximum(_ln(t, g_ref[...], bt_ref[...]), 0.0)
    o_ref[...] = jnp.dot(t, w2_ref[...],
                         preferred_element_type=jnp.float32) + b2_ref[...]


def _full(shape):
    return pl.BlockSpec(shape, lambda *i: tuple(0 for _ in shape))


def _dense0(x, w1, b1, g, beta, w2, b2):
    n, din = x.shape
    hid = w1.shape[1]
    dout = w2.shape[1]
    bs = 1000
    return pl.pallas_call(
        _dense0_body,
        grid=(n // bs,),
        in_specs=[pl.BlockSpec((bs, din), lambda i: (i, 0)),
                  _full((din, hid)), _full((1, hid)), _full((1, hid)),
                  _full((1, hid)), _full((hid, dout)), _full((1, dout))],
        out_specs=pl.BlockSpec((bs, dout), lambda i: (i, 0)),
        out_shape=jax.ShapeDtypeStruct((n, dout), jnp.float32),
    )(x, w1, b1.reshape(1, -1), g.reshape(1, -1), beta.reshape(1, -1),
      w2, b2.reshape(1, -1))


def _dense1(a, b, w1, b1, g, beta, w2, b2):
    n, din = a.shape
    hid = w1.shape[1]
    dout = w2.shape[1]
    bs = 1000
    w1a = w1[:din]
    w1b = w1[din:]
    return pl.pallas_call(
        _dense1_body,
        grid=(n // bs,),
        in_specs=[pl.BlockSpec((bs, din), lambda i: (i, 0)),
                  pl.BlockSpec((bs, din), lambda i: (i, 0)),
                  _full((din, hid)), _full((din, hid)), _full((1, hid)),
                  _full((1, hid)), _full((1, hid)), _full((hid, dout)),
                  _full((1, dout))],
        out_specs=pl.BlockSpec((bs, dout), lambda i: (i, 0)),
        out_shape=jax.ShapeDtypeStruct((n, dout), jnp.float32),
    )(a, b, w1a, w1b, b1.reshape(1, -1), g.reshape(1, -1),
      beta.reshape(1, -1), w2, b2.reshape(1, -1))


def _pool_body(h_ref, a_ref, b_ref, o_ref):
    o_ref[...] = jnp.concatenate(
        [jnp.max(h_ref[...], axis=1), jnp.max(a_ref[...], axis=1),
         jnp.max(b_ref[...], axis=1)], axis=1)


def _pool(h, a, b, group):
    n, d = h.shape
    da = a.shape[1]
    nc = n // group
    hr = h.reshape(nc, group, d)
    ar = a.reshape(nc, group, da)
    br = b.reshape(nc, group, da)
    bs = 200
    return pl.pallas_call(
        _pool_body,
        grid=(nc // bs,),
        in_specs=[pl.BlockSpec((bs, group, d), lambda i: (i, 0, 0)),
                  pl.BlockSpec((bs, group, da), lambda i: (i, 0, 0)),
                  pl.BlockSpec((bs, group, da), lambda i: (i, 0, 0))],
        out_specs=pl.BlockSpec((bs, d + 2 * da), lambda i: (i, 0)),
        out_shape=jax.ShapeDtypeStruct((nc, d + 2 * da), jnp.float32),
    )(hr, ar, br)


def _tail_body(p_ref, vl_ref, qw_ref, qb_ref, kw_ref, kb_ref, vw_ref, vb_ref,
               pw1_ref, pb1_ref, pg_ref, pbt_ref, pw2_ref, pb2_ref, o_ref,
               *, batch, t):
    pooled = p_ref[...]
    nc = pooled.shape[0]
    ssq = jnp.sum(pooled * pooled, axis=0, keepdims=True)
    pn = pooled / jnp.sqrt(ssq)
    q = jnp.dot(pn, qw_ref[...], preferred_element_type=jnp.float32) + qb_ref[...]
    k = jnp.dot(pn, kw_ref[...], preferred_element_type=jnp.float32) + kb_ref[...]
    v = jnp.dot(pn, vw_ref[...], preferred_element_type=jnp.float32) + vb_ref[...]
    cols = lax.broadcasted_iota(jnp.int32, (batch, nc), 1)
    bidx = lax.broadcasted_iota(jnp.int32, (batch, nc), 0)
    sel = (cols == bidx * t).astype(jnp.float32)
    q0 = jnp.dot(sel, q, preferred_element_type=jnp.float32)
    scores = lax.dot_general(q0, k, (((1,), (1,)), ((), ())),
                             preferred_element_type=jnp.float32)
    seg = (cols // t) == bidx
    s_in = cols % t
    keep = s_in <= vl_ref[...]
    sc = jnp.where(seg & keep, scores,
                   jnp.where(seg, jnp.float32(-1e-6), -jnp.inf))
    mx = jnp.max(sc, axis=-1, keepdims=True)
    e = jnp.exp(sc - mx)
    attn = e / jnp.sum(e, axis=-1, keepdims=True)
    h0 = jnp.dot(attn, v, preferred_element_type=jnp.float32)
    p = jnp.dot(h0, pw1_ref[...], preferred_element_type=jnp.float32) + pb1_ref[...]
    p = jnp.maximum(_ln(p, pg_ref[...], pbt_ref[...]), 0.0)
    o_ref[...] = jnp.dot(p, pw2_ref[...],
                         preferred_element_type=jnp.float32) + pb2_ref[...]


def _tail(pooled, valid_len, t, q_w, q_b, k_w, k_b, v_w, v_b,
          pred_w1, pred_b1, pred_g, pred_beta, pred_w2, pred_b2):
    nc, p = pooled.shape
    batch = nc // t
    ggw = q_w.shape[1]
    out = pred_w2.shape[1]
    pw = pred_w1.shape[1]
    return pl.pallas_call(
        functools.partial(_tail_body, batch=batch, t=t),
        in_specs=[_full((nc, p)), _full((batch, 1)),
                  _full((p, ggw)), _full((1, ggw)),
                  _full((p, ggw)), _full((1, ggw)),
                  _full((p, ggw)), _full((1, ggw)),
                  _full((ggw, pw)), _full((1, pw)), _full((1, pw)),
                  _full((1, pw)), _full((pw, out)), _full((1, out))],
        out_specs=_full((batch, out)),
        out_shape=jax.ShapeDtypeStruct((batch, out), jnp.float32),
    )(pooled, valid_len.reshape(batch, 1),
      q_w, q_b.reshape(1, -1), k_w, k_b.reshape(1, -1),
      v_w, v_b.reshape(1, -1), pred_w1, pred_b1.reshape(1, -1),
      pred_g.reshape(1, -1), pred_beta.reshape(1, -1),
      pred_w2, pred_b2.reshape(1, -1))


def kernel(x, edge_index, cluster, valid_len, time_step_len,
           gl0_w1, gl0_b1, gl0_g, gl0_beta, gl0_w2, gl0_b2,
           gl1_w1, gl1_b1, gl1_g, gl1_beta, gl1_w2, gl1_b2,
           q_w, q_b, k_w, k_b, v_w, v_b,
           pred_w1, pred_b1, pred_g, pred_beta, pred_w2, pred_b2):
    src = edge_index[0]
    dst = edge_index[1]
    n = x.shape[0]

    h0 = _dense0(x, gl0_w1, gl0_b1, gl0_g, gl0_beta, gl0_w2, gl0_b2)
    agg0 = _segment_max(h0, src, dst)
    h1 = _dense1(h0, agg0, gl1_w1, gl1_b1, gl1_g, gl1_beta, gl1_w2, gl1_b2)
    half = h1.shape[1] // 2
    agg1a = _segment_max(h1[:, :half], src, dst)
    agg1b = _segment_max(h1[:, half:], src, dst)

    t = 50
    num_clusters = valid_len.shape[0] * t
    group = n // num_clusters
    pooled = _pool(h1, agg1a, agg1b, group)
    return _tail(pooled, valid_len, t, q_w, q_b, k_w, k_b, v_w, v_b,
                 pred_w1, pred_b1, pred_g, pred_beta, pred_w2, pred_b2)


# speedup vs baseline: 10.1889x; 1.3158x over previous
import functools

import jax
import jax.numpy as jnp
from jax import lax
from jax.experimental import pallas as pl
from jax.experimental.pallas import tpu as pltpu
from jax.experimental.pallas import tpu_sc as plsc

_LANES = 16
_C = 1600
_G = 64
_UNROLL = 4
_LIST = 1664


@functools.lru_cache(maxsize=None)
def _make_segmax(n_nodes, n_edges, d):
    info = plsc.get_sparse_core_info()
    nw = info.num_cores * info.num_subcores
    rpw = ((n_nodes + nw * 8 - 1) // (nw * 8)) * 8
    npad = rpw * nw
    nchunk = n_edges // _C
    assert nchunk % 2 == 0
    mesh = plsc.VectorSubcoreMesh(core_axis_name="c", subcore_axis_name="s")

    @functools.partial(
        pl.kernel,
        out_type=jax.ShapeDtypeStruct((npad, d), jnp.float32),
        mesh=mesh,
        compiler_params=pltpu.CompilerParams(needs_layout_passes=False),
        scratch_types=[
            pltpu.VMEM((rpw + 8, d), jnp.float32),
            pltpu.VMEM((_G, d), jnp.float32),
            pltpu.VMEM((_G, d), jnp.float32),
            pltpu.VMEM((_C,), jnp.int32),
            pltpu.VMEM((_C,), jnp.int32),
            pltpu.VMEM((_LIST,), jnp.int32),
            pltpu.VMEM((_LIST,), jnp.int32),
            pltpu.VMEM((_LIST,), jnp.int32),
            pltpu.VMEM((_LIST,), jnp.int32),
            pltpu.SemaphoreType.DMA,
            pltpu.SemaphoreType.DMA,
            pltpu.SemaphoreType.DMA,
        ],
    )
    def segmax(h_hbm, src_hbm, dst_hbm, out_hbm,
               agg, rows_a, rows_b, srcc, dstc,
               lsrc_a, lld_a, lsrc_b, lld_b, sem_a, sem_b, sem_x):
        wid = lax.axis_index("s") * info.num_cores + lax.axis_index("c")
        base = wid * rpw
        neg = jnp.full((_LANES,), -jnp.inf, jnp.float32)

        def init_body(r, carry):
            for j in range(d // _LANES):
                agg[r, pl.ds(j * _LANES, _LANES)] = neg
            return carry
        lax.fori_loop(0, rpw, init_body, 0)

        zero = jnp.zeros((_LANES,), jnp.int32)

        def zero_body(i, carry):
            lsrc_a[pl.ds(i * _LANES, _LANES)] = zero
            lsrc_b[pl.ds(i * _LANES, _LANES)] = zero
            return carry
        lax.fori_loop(0, _LIST // _LANES, zero_body, 0)

        dummy = jnp.full((_LANES,), rpw, jnp.int32)

        def front(c, lsrc, lld, rows, sem):
            pltpu.sync_copy(src_hbm.at[pl.ds(c * _C, _C)], srcc)
            pltpu.sync_copy(dst_hbm.at[pl.ds(c * _C, _C)], dstc)

            def compact_body(i, cnt_vec):
                subs = []
                for k in range(_UNROLL):
                    off = (i * _UNROLL + k) * _LANES
                    d16 = dstc[pl.ds(off, _LANES)]
                    s16 = srcc[pl.ds(off, _LANES)]
                    m = (d16 >= base) & (d16 < base + rpw)
                    pref = plsc.cumsum(m.astype(jnp.int32))
                    pop = plsc.all_reduce_population_count(m)
                    subs.append((d16, s16, m, pref, pop))
                acc = cnt_vec
                for d16, s16, m, pref, pop in subs:
                    idx = acc + pref - 1
                    plsc.store_scatter(lsrc, [idx], s16, mask=m)
                    plsc.store_scatter(lld, [idx], d16 - base, mask=m)
                    acc = acc + pop
                return acc

            cnt_vec = lax.fori_loop(0, _C // (_LANES * _UNROLL), compact_body,
                                    jnp.zeros((_LANES,), jnp.int32))
            cnt = jnp.max(cnt_vec)
            pad_idx = cnt + lax.iota(jnp.int32, _LANES)
            plsc.store_scatter(lld, [pad_idx], dummy)
            pltpu.async_copy(h_hbm.at[lsrc.at[pl.ds(0, _G)]], rows, sem)
            return cnt

        def rmw_batch(lld, rows, lo, g_lo, g_hi):
            def group_body(g, carry2):
                ld_vec = lld[pl.ds(g * _LANES, _LANES)]
                rbase = g * _LANES - lo
                for l in range(_LANES):
                    ld = ld_vec[l]
                    rb = rbase + l
                    for j in range(d // _LANES):
                        sl = pl.ds(j * _LANES, _LANES)
                        agg[ld, sl] = jnp.maximum(agg[ld, sl], rows[rb, sl])
                return carry2
            lax.fori_loop(g_lo, g_hi, group_body, 0)

        def drain(cnt, lsrc, lld, rows, sem):
            pltpu.make_async_copy(h_hbm.at[pl.ds(0, _G)], rows, sem).wait()
            ng = (cnt + 15) >> 4
            gpb = _G // _LANES
            rmw_batch(lld, rows, 0, 0, jnp.minimum(ng, gpb))

            def batch_body(b, carry):
                lo = b * _G
                pltpu.async_copy(h_hbm.at[lsrc.at[pl.ds(lo, _G)]],
                                 rows, sem).wait()
                rmw_batch(lld, rows, lo, b * gpb,
                          jnp.minimum(ng, (b + 1) * gpb))
                return carry
            nb = (cnt + _G - 1) >> 6
            lax.fori_loop(1, nb, batch_body, 0)

        cnt_a0 = front(0, lsrc_a, lld_a, rows_a, sem_a)

        def pipe_body(t, cnt_a):
            cnt_b = front(2 * t + 1, lsrc_b, lld_b, rows_b, sem_b)
            drain(cnt_a, lsrc_a, lld_a, rows_a, sem_a)
            cnt_a2 = front(2 * t + 2, lsrc_a, lld_a, rows_a, sem_a)
            drain(cnt_b, lsrc_b, lld_b, rows_b, sem_b)
            return cnt_a2

        cnt_last = lax.fori_loop(0, nchunk // 2 - 1, pipe_body, cnt_a0)
        cnt_b_last = front(nchunk - 1, lsrc_b, lld_b, rows_b, sem_b)
        drain(cnt_last, lsrc_a, lld_a, rows_a, sem_a)
        drain(cnt_b_last, lsrc_b, lld_b, rows_b, sem_b)

        def fin_body(r, carry):
            for j in range(d // _LANES):
                sl = pl.ds(j * _LANES, _LANES)
                v = agg[r, sl]
                agg[r, sl] = jnp.where(v == -jnp.inf, 0.0, v)
            return carry
        lax.fori_loop(0, rpw, fin_body, 0)
        pltpu.sync_copy(agg.at[pl.ds(0, rpw)], out_hbm.at[pl.ds(base, rpw)])

    return segmax, npad


def _segment_max(h, src, dst):
    n, d = h.shape
    fn, npad = _make_segmax(n, src.shape[0], d)
    return fn(h, src, dst)[:n]


def _ln(t, g, beta):
    m = jnp.mean(t, axis=-1, keepdims=True)
    v = jnp.mean((t - m) ** 2, axis=-1, keepdims=True)
    return (t - m) / jnp.sqrt(v + 1e-5) * g + beta


def _dense0_body(x_ref, w1_ref, b1_ref, g_ref, bt_ref, w2_ref, b2_ref, o_ref):
    t = jnp.dot(x_ref[...], w1_ref[...],
                preferred_element_type=jnp.float32) + b1_ref[...]
    t = jnp.maximum(_ln(t, g_ref[...], bt_ref[...]), 0.0)
    o_ref[...] = jnp.dot(t, w2_ref[...],
                         preferred_element_type=jnp.float32) + b2_ref[...]


def _dense1_body(a_ref, b_ref, w1a_ref, w1b_ref, b1_ref, g_ref, bt_ref,
                 w2_ref, b2_ref, o_ref):
    t = (jnp.dot(a_ref[...], w1a_ref[...], preferred_element_type=jnp.float32)
         + jnp.dot(b_ref[...], w1b_ref[...],
                   preferred_element_type=jnp.float32) + b1_ref[...])
    t = jnp.maximum(_ln(t, g_ref[...], bt_ref[...]), 0.0)
    o_ref[...] = jnp.dot(t, w2_ref[...],
                         preferred_element_type=jnp.float32) + b2_ref[...]


def _full(shape):
    return pl.BlockSpec(shape, lambda *i: tuple(0 for _ in shape))


def _dense0(x, w1, b1, g, beta, w2, b2):
    n, din = x.shape
    hid = w1.shape[1]
    dout = w2.shape[1]
    bs = 1000
    return pl.pallas_call(
        _dense0_body,
        grid=(n // bs,),
        in_specs=[pl.BlockSpec((bs, din), lambda i: (i, 0)),
                  _full((din, hid)), _full((1, hid)), _full((1, hid)),
                  _full((1, hid)), _full((hid, dout)), _full((1, dout))],
        out_specs=pl.BlockSpec((bs, dout), lambda i: (i, 0)),
        out_shape=jax.ShapeDtypeStruct((n, dout), jnp.float32),
    )(x, w1, b1.reshape(1, -1), g.reshape(1, -1), beta.reshape(1, -1),
      w2, b2.reshape(1, -1))


def _dense1(a, b, w1, b1, g, beta, w2, b2):
    n, din = a.shape
    hid = w1.shape[1]
    dout = w2.shape[1]
    bs = 1000
    w1a = w1[:din]
    w1b = w1[din:]
    return pl.pallas_call(
        _dense1_body,
        grid=(n // bs,),
        in_specs=[pl.BlockSpec((bs, din), lambda i: (i, 0)),
                  pl.BlockSpec((bs, din), lambda i: (i, 0)),
                  _full((din, hid)), _full((din, hid)), _full((1, hid)),
                  _full((1, hid)), _full((1, hid)), _full((hid, dout)),
                  _full((1, dout))],
        out_specs=pl.BlockSpec((bs, dout), lambda i: (i, 0)),
        out_shape=jax.ShapeDtypeStruct((n, dout), jnp.float32),
    )(a, b, w1a, w1b, b1.reshape(1, -1), g.reshape(1, -1),
      beta.reshape(1, -1), w2, b2.reshape(1, -1))


def _pool_body(h_ref, a_ref, o_ref):
    left = jnp.max(h_ref[...], axis=1)
    right = jnp.max(a_ref[...], axis=1)
    o_ref[...] = jnp.concatenate([left, right], axis=1)


def _pool(h, a, group):
    n, d = h.shape
    nc = n // group
    hr = h.reshape(nc, group, d)
    ar = a.reshape(nc, group, d)
    bs = 200
    return pl.pallas_call(
        _pool_body,
        grid=(nc // bs,),
        in_specs=[pl.BlockSpec((bs, group, d), lambda i: (i, 0, 0)),
                  pl.BlockSpec((bs, group, d), lambda i: (i, 0, 0))],
        out_specs=pl.BlockSpec((bs, 2 * d), lambda i: (i, 0)),
        out_shape=jax.ShapeDtypeStruct((nc, 2 * d), jnp.float32),
    )(hr, ar)


def _tail_body(p_ref, vl_ref, qw_ref, qb_ref, kw_ref, kb_ref, vw_ref, vb_ref,
               pw1_ref, pb1_ref, pg_ref, pbt_ref, pw2_ref, pb2_ref, o_ref,
               *, batch, t):
    pooled = p_ref[...]
    nc = pooled.shape[0]
    ssq = jnp.sum(pooled * pooled, axis=0, keepdims=True)
    pn = pooled / jnp.sqrt(ssq)
    q = jnp.dot(pn, qw_ref[...], preferred_element_type=jnp.float32) + qb_ref[...]
    k = jnp.dot(pn, kw_ref[...], preferred_element_type=jnp.float32) + kb_ref[...]
    v = jnp.dot(pn, vw_ref[...], preferred_element_type=jnp.float32) + vb_ref[...]
    cols = lax.broadcasted_iota(jnp.int32, (batch, nc), 1)
    bidx = lax.broadcasted_iota(jnp.int32, (batch, nc), 0)
    sel = (cols == bidx * t).astype(jnp.float32)
    q0 = jnp.dot(sel, q, preferred_element_type=jnp.float32)
    scores = lax.dot_general(q0, k, (((1,), (1,)), ((), ())),
                             preferred_element_type=jnp.float32)
    seg = (cols // t) == bidx
    s_in = cols % t
    keep = s_in <= vl_ref[...]
    sc = jnp.where(seg & keep, scores,
                   jnp.where(seg, jnp.float32(-1e-6), -jnp.inf))
    mx = jnp.max(sc, axis=-1, keepdims=True)
    e = jnp.exp(sc - mx)
    attn = e / jnp.sum(e, axis=-1, keepdims=True)
    h0 = jnp.dot(attn, v, preferred_element_type=jnp.float32)
    p = jnp.dot(h0, pw1_ref[...], preferred_element_type=jnp.float32) + pb1_ref[...]
    p = jnp.maximum(_ln(p, pg_ref[...], pbt_ref[...]), 0.0)
    o_ref[...] = jnp.dot(p, pw2_ref[...],
                         preferred_element_type=jnp.float32) + pb2_ref[...]


def _tail(pooled, valid_len, t, q_w, q_b, k_w, k_b, v_w, v_b,
          pred_w1, pred_b1, pred_g, pred_beta, pred_w2, pred_b2):
    nc, p = pooled.shape
    batch = nc // t
    ggw = q_w.shape[1]
    out = pred_w2.shape[1]
    pw = pred_w1.shape[1]
    return pl.pallas_call(
        functools.partial(_tail_body, batch=batch, t=t),
        in_specs=[_full((nc, p)), _full((batch, 1)),
                  _full((p, ggw)), _full((1, ggw)),
                  _full((p, ggw)), _full((1, ggw)),
                  _full((p, ggw)), _full((1, ggw)),
                  _full((ggw, pw)), _full((1, pw)), _full((1, pw)),
                  _full((1, pw)), _full((pw, out)), _full((1, out))],
        out_specs=_full((batch, out)),
        out_shape=jax.ShapeDtypeStruct((batch, out), jnp.float32),
    )(pooled, valid_len.reshape(batch, 1),
      q_w, q_b.reshape(1, -1), k_w, k_b.reshape(1, -1),
      v_w, v_b.reshape(1, -1), pred_w1, pred_b1.reshape(1, -1),
      pred_g.reshape(1, -1), pred_beta.reshape(1, -1),
      pred_w2, pred_b2.reshape(1, -1))


def kernel(x, edge_index, cluster, valid_len, time_step_len,
           gl0_w1, gl0_b1, gl0_g, gl0_beta, gl0_w2, gl0_b2,
           gl1_w1, gl1_b1, gl1_g, gl1_beta, gl1_w2, gl1_b2,
           q_w, q_b, k_w, k_b, v_w, v_b,
           pred_w1, pred_b1, pred_g, pred_beta, pred_w2, pred_b2):
    src = edge_index[0]
    dst = edge_index[1]
    n = x.shape[0]

    h0 = _dense0(x, gl0_w1, gl0_b1, gl0_g, gl0_beta, gl0_w2, gl0_b2)
    agg0 = _segment_max(h0, src, dst)
    h1 = _dense1(h0, agg0, gl1_w1, gl1_b1, gl1_g, gl1_beta, gl1_w2, gl1_b2)
    agg1 = _segment_max(h1, src, dst)

    t = 50
    num_clusters = valid_len.shape[0] * t
    group = n // num_clusters
    pooled = _pool(h1, agg1, group)
    return _tail(pooled, valid_len, t, q_w, q_b, k_w, k_b, v_w, v_b,
                 pred_w1, pred_b1, pred_g, pred_beta, pred_w2, pred_b2)


# speedup vs baseline: 19.3983x; 1.9039x over previous
import functools

import jax
import jax.numpy as jnp
from jax import lax
from jax.experimental import pallas as pl
from jax.experimental.pallas import tpu as pltpu
from jax.experimental.pallas import tpu_sc as plsc

_LANES = 16


@functools.lru_cache(maxsize=None)
def _make_segmax(n_nodes, n_edges, d):
    info = plsc.get_sparse_core_info()
    nw = info.num_cores * info.num_subcores
    rpw = ((n_nodes + nw * 8 - 1) // (nw * 8)) * 8
    npad = rpw * nw
    if d <= 128:
        C, G, GSH = 3200, 128, 7
    else:
        C, G, GSH = 2000, 64, 6
    UN = 125 if d > 128 else 8
    LIST = C + G
    nchunk = n_edges // C
    assert nchunk % 2 == 0
    gpb = G // _LANES
    mesh = plsc.VectorSubcoreMesh(core_axis_name="c", subcore_axis_name="s")

    @functools.partial(
        pl.kernel,
        out_type=jax.ShapeDtypeStruct((npad, d), jnp.float32),
        mesh=mesh,
        compiler_params=pltpu.CompilerParams(needs_layout_passes=False),
        scratch_types=[
            pltpu.VMEM((rpw + 8, d), jnp.float32),
            pltpu.VMEM((G, d), jnp.float32),
            pltpu.VMEM((G, d), jnp.float32),
            pltpu.VMEM((C,), jnp.int32),
            pltpu.VMEM((C,), jnp.int32),
            pltpu.VMEM((C,), jnp.int32),
            pltpu.VMEM((C,), jnp.int32),
            pltpu.VMEM((LIST,), jnp.int32),
            pltpu.VMEM((LIST,), jnp.int32),
            pltpu.VMEM((LIST,), jnp.int32),
            pltpu.VMEM((LIST,), jnp.int32),
            pltpu.SemaphoreType.DMA,
            pltpu.SemaphoreType.DMA,
            pltpu.SemaphoreType.DMA,
            pltpu.SemaphoreType.DMA,
        ],
    )
    def segmax(h_hbm, src_hbm, dst_hbm, out_hbm,
               agg, rows_a, rows_b, srcc_a, dstc_a, srcc_b, dstc_b,
               lsrc_a, lld_a, lsrc_b, lld_b, sem_a, sem_b, sxa, sxb):
        wid = lax.axis_index("s") * info.num_cores + lax.axis_index("c")
        base = wid * rpw
        neg = jnp.full((_LANES,), -jnp.inf, jnp.float32)

        def init_body(r, carry):
            for j in range(d // _LANES):
                agg[r, pl.ds(j * _LANES, _LANES)] = neg
            return carry
        lax.fori_loop(0, rpw, init_body, 0)

        zero = jnp.zeros((_LANES,), jnp.int32)

        def zero_body(i, carry):
            lsrc_a[pl.ds(i * _LANES, _LANES)] = zero
            lsrc_b[pl.ds(i * _LANES, _LANES)] = zero
            return carry
        lax.fori_loop(0, LIST // _LANES, zero_body, 0)

        dummy = jnp.full((_LANES,), rpw, jnp.int32)

        def fetch_idx(c, srcc, dstc, sx):
            pltpu.async_copy(src_hbm.at[pl.ds(c * C, C)], srcc, sx)
            pltpu.async_copy(dst_hbm.at[pl.ds(c * C, C)], dstc, sx)

        def wait_idx(srcc, dstc, sx):
            pltpu.make_async_copy(src_hbm.at[pl.ds(0, C)], srcc, sx).wait()
            pltpu.make_async_copy(dst_hbm.at[pl.ds(0, C)], dstc, sx).wait()

        def front(c, srcc, dstc, sx, lsrc, lld, rows, sem):
            wait_idx(srcc, dstc, sx)

            def compact_body(i, cnt_vec):
                subs = []
                for k in range(UN):
                    off = (i * UN + k) * _LANES
                    d16 = dstc[pl.ds(off, _LANES)]
                    s16 = srcc[pl.ds(off, _LANES)]
                    m = (d16 >= base) & (d16 < base + rpw)
                    pref = plsc.cumsum(m.astype(jnp.int32))
                    pop = plsc.all_reduce_population_count(m)
                    subs.append((d16, s16, m, pref, pop))
                acc = cnt_vec
                for d16, s16, m, pref, pop in subs:
                    idx = acc + pref - 1
                    plsc.store_scatter(lsrc, [idx], s16, mask=m)
                    plsc.store_scatter(lld, [idx], d16 - base, mask=m)
                    acc = acc + pop
                return acc

            cnt_vec = lax.fori_loop(0, C // (_LANES * UN), compact_body,
                                    jnp.zeros((_LANES,), jnp.int32))

            @pl.when(c + 2 < nchunk)
            def _():
                fetch_idx(c + 2, srcc, dstc, sx)

            cnt = jnp.max(cnt_vec)
            pad_idx = cnt + lax.iota(jnp.int32, _LANES)
            plsc.store_scatter(lld, [pad_idx], dummy)
            for k in range(G // 8):
                @pl.when(cnt > k * 8)
                def _():
                    pltpu.async_copy(
                        h_hbm.at[lsrc.at[pl.ds(k * 8, 8)]],
                        rows.at[pl.ds(k * 8, 8)], sem)
            return cnt

        def drain(cnt, lsrc, lld, rows, sem):
            ng = (cnt + 15) >> 4
            nb = (cnt + G - 1) >> GSH

            def batch_body(b, carry):
                for k in range(G // 8):
                    @pl.when((b > 0) & (cnt > b * G + k * 8))
                    def _():
                        pltpu.async_copy(
                            h_hbm.at[lsrc.at[pl.ds(b * G + k * 8, 8)]],
                            rows.at[pl.ds(k * 8, 8)], sem)
                for k in range(G // 8):
                    @pl.when(cnt > b * G + k * 8)
                    def _():
                        pltpu.make_async_copy(
                            h_hbm.at[pl.ds(0, 8)],
                            rows.at[pl.ds(k * 8, 8)], sem).wait()
                lo = b * G

                def group_body(g, carry2):
                    ld_vec = lld[pl.ds(g * _LANES, _LANES)]
                    rbase = g * _LANES - lo
                    for l in range(_LANES):
                        ld = ld_vec[l]
                        rb = rbase + l
                        for j in range(d // _LANES):
                            sl = pl.ds(j * _LANES, _LANES)
                            agg[ld, sl] = jnp.maximum(agg[ld, sl],
                                                      rows[rb, sl])
                    return carry2
                lax.fori_loop(b * gpb, jnp.minimum(ng, (b + 1) * gpb),
                              group_body, 0)
                return carry
            lax.fori_loop(0, jnp.maximum(nb, 1), batch_body, 0)

        fetch_idx(0, srcc_a, dstc_a, sxa)
        fetch_idx(1, srcc_b, dstc_b, sxb)
        cnt_a0 = front(0, srcc_a, dstc_a, sxa, lsrc_a, lld_a, rows_a, sem_a)

        def pipe_body(t, cnt_a):
            c = 2 * t + 1
            cnt_b = front(c, srcc_b, dstc_b, sxb, lsrc_b, lld_b, rows_b,
                          sem_b)
            drain(cnt_a, lsrc_a, lld_a, rows_a, sem_a)
            cnt_a2 = front(c + 1, srcc_a, dstc_a, sxa, lsrc_a, lld_a,
                           rows_a, sem_a)
            drain(cnt_b, lsrc_b, lld_b, rows_b, sem_b)
            return cnt_a2

        cnt_last = lax.fori_loop(0, nchunk // 2 - 1, pipe_body, cnt_a0)
        cnt_b = front(nchunk - 1, srcc_b, dstc_b, sxb, lsrc_b, lld_b,
                      rows_b, sem_b)
        drain(cnt_last, lsrc_a, lld_a, rows_a, sem_a)
        drain(cnt_b, lsrc_b, lld_b, rows_b, sem_b)

        def fin_body(r, carry):
            for j in range(d // _LANES):
                sl = pl.ds(j * _LANES, _LANES)
                v = agg[r, sl]
                agg[r, sl] = jnp.where(v == -jnp.inf, 0.0, v)
            return carry
        lax.fori_loop(0, rpw, fin_body, 0)
        pltpu.sync_copy(agg.at[pl.ds(0, rpw)], out_hbm.at[pl.ds(base, rpw)])

    return segmax, npad


def _segment_max(h, src, dst):
    n, d = h.shape
    fn, npad = _make_segmax(n, src.shape[0], d)
    return fn(h, src, dst)[:n]


def _ln(t, g, beta):
    m = jnp.mean(t, axis=-1, keepdims=True)
    v = jnp.mean((t - m) ** 2, axis=-1, keepdims=True)
    return (t - m) / jnp.sqrt(v + 1e-5) * g + beta


def _dense0_body(x_ref, w1_ref, b1_ref, g_ref, bt_ref, w2_ref, b2_ref, o_ref):
    t = jnp.dot(x_ref[...], w1_ref[...],
                preferred_element_type=jnp.float32) + b1_ref[...]
    t = jnp.maximum(_ln(t, g_ref[...], bt_ref[...]), 0.0)
    o_ref[...] = jnp.dot(t, w2_ref[...],
                         preferred_element_type=jnp.float32) + b2_ref[...]


def _dense1_body(a_ref, b_ref, w1a_ref, w1b_ref, b1_ref, g_ref, bt_ref,
                 w2_ref, b2_ref, o_ref):
    t = (jnp.dot(a_ref[...], w1a_ref[...], preferred_element_type=jnp.float32)
         + jnp.dot(b_ref[...], w1b_ref[...],
                   preferred_element_type=jnp.float32) + b1_ref[...])
    t = jnp.maximum(_ln(t, g_ref[...], bt_ref[...]), 0.0)
    o_ref[...] = jnp.dot(t, w2_ref[...],
                         preferred_element_type=jnp.float32) + b2_ref[...]


def _full(shape):
    return pl.BlockSpec(shape, lambda *i: tuple(0 for _ in shape))


def _dense0(x, w1, b1, g, beta, w2, b2):
    n, din = x.shape
    hid = w1.shape[1]
    dout = w2.shape[1]
    bs = 1000
    return pl.pallas_call(
        _dense0_body,
        grid=(n // bs,),
        in_specs=[pl.BlockSpec((bs, din), lambda i: (i, 0)),
                  _full((din, hid)), _full((1, hid)), _full((1, hid)),
                  _full((1, hid)), _full((hid, dout)), _full((1, dout))],
        out_specs=pl.BlockSpec((bs, dout), lambda i: (i, 0)),
        out_shape=jax.ShapeDtypeStruct((n, dout), jnp.float32),
    )(x, w1, b1.reshape(1, -1), g.reshape(1, -1), beta.reshape(1, -1),
      w2, b2.reshape(1, -1))


def _dense1(a, b, w1, b1, g, beta, w2, b2):
    n, din = a.shape
    hid = w1.shape[1]
    dout = w2.shape[1]
    bs = 1000
    w1a = w1[:din]
    w1b = w1[din:]
    return pl.pallas_call(
        _dense1_body,
        grid=(n // bs,),
        in_specs=[pl.BlockSpec((bs, din), lambda i: (i, 0)),
                  pl.BlockSpec((bs, din), lambda i: (i, 0)),
                  _full((din, hid)), _full((din, hid)), _full((1, hid)),
                  _full((1, hid)), _full((1, hid)), _full((hid, dout)),
                  _full((1, dout))],
        out_specs=pl.BlockSpec((bs, dout), lambda i: (i, 0)),
        out_shape=jax.ShapeDtypeStruct((n, dout), jnp.float32),
    )(a, b, w1a, w1b, b1.reshape(1, -1), g.reshape(1, -1),
      beta.reshape(1, -1), w2, b2.reshape(1, -1))


def _pool_body(h_ref, a_ref, o_ref):
    left = jnp.max(h_ref[...], axis=1)
    right = jnp.max(a_ref[...], axis=1)
    o_ref[...] = jnp.concatenate([left, right], axis=1)


def _pool(h, a, group):
    n, d = h.shape
    nc = n // group
    hr = h.reshape(nc, group, d)
    ar = a.reshape(nc, group, d)
    bs = 200
    return pl.pallas_call(
        _pool_body,
        grid=(nc // bs,),
        in_specs=[pl.BlockSpec((bs, group, d), lambda i: (i, 0, 0)),
                  pl.BlockSpec((bs, group, d), lambda i: (i, 0, 0))],
        out_specs=pl.BlockSpec((bs, 2 * d), lambda i: (i, 0)),
        out_shape=jax.ShapeDtypeStruct((nc, 2 * d), jnp.float32),
    )(hr, ar)


def _tail_body(p_ref, vl_ref, qw_ref, qb_ref, kw_ref, kb_ref, vw_ref, vb_ref,
               pw1_ref, pb1_ref, pg_ref, pbt_ref, pw2_ref, pb2_ref, o_ref,
               *, batch, t):
    pooled = p_ref[...]
    nc = pooled.shape[0]
    ssq = jnp.sum(pooled * pooled, axis=0, keepdims=True)
    pn = pooled / jnp.sqrt(ssq)
    q = jnp.dot(pn, qw_ref[...], preferred_element_type=jnp.float32) + qb_ref[...]
    k = jnp.dot(pn, kw_ref[...], preferred_element_type=jnp.float32) + kb_ref[...]
    v = jnp.dot(pn, vw_ref[...], preferred_element_type=jnp.float32) + vb_ref[...]
    cols = lax.broadcasted_iota(jnp.int32, (batch, nc), 1)
    bidx = lax.broadcasted_iota(jnp.int32, (batch, nc), 0)
    sel = (cols == bidx * t).astype(jnp.float32)
    q0 = jnp.dot(sel, q, preferred_element_type=jnp.float32)
    scores = lax.dot_general(q0, k, (((1,), (1,)), ((), ())),
                             preferred_element_type=jnp.float32)
    seg = (cols // t) == bidx
    s_in = cols % t
    keep = s_in <= vl_ref[...]
    sc = jnp.where(seg & keep, scores,
                   jnp.where(seg, jnp.float32(-1e-6), -jnp.inf))
    mx = jnp.max(sc, axis=-1, keepdims=True)
    e = jnp.exp(sc - mx)
    attn = e / jnp.sum(e, axis=-1, keepdims=True)
    h0 = jnp.dot(attn, v, preferred_element_type=jnp.float32)
    p = jnp.dot(h0, pw1_ref[...], preferred_element_type=jnp.float32) + pb1_ref[...]
    p = jnp.maximum(_ln(p, pg_ref[...], pbt_ref[...]), 0.0)
    o_ref[...] = jnp.dot(p, pw2_ref[...],
                         preferred_element_type=jnp.float32) + pb2_ref[...]


def _tail(pooled, valid_len, t, q_w, q_b, k_w, k_b, v_w, v_b,
          pred_w1, pred_b1, pred_g, pred_beta, pred_w2, pred_b2):
    nc, p = pooled.shape
    batch = nc // t
    ggw = q_w.shape[1]
    out = pred_w2.shape[1]
    pw = pred_w1.shape[1]
    return pl.pallas_call(
        functools.partial(_tail_body, batch=batch, t=t),
        in_specs=[_full((nc, p)), _full((batch, 1)),
                  _full((p, ggw)), _full((1, ggw)),
                  _full((p, ggw)), _full((1, ggw)),
                  _full((p, ggw)), _full((1, ggw)),
                  _full((ggw, pw)), _full((1, pw)), _full((1, pw)),
                  _full((1, pw)), _full((pw, out)), _full((1, out))],
        out_specs=_full((batch, out)),
        out_shape=jax.ShapeDtypeStruct((batch, out), jnp.float32),
    )(pooled, valid_len.reshape(batch, 1),
      q_w, q_b.reshape(1, -1), k_w, k_b.reshape(1, -1),
      v_w, v_b.reshape(1, -1), pred_w1, pred_b1.reshape(1, -1),
      pred_g.reshape(1, -1), pred_beta.reshape(1, -1),
      pred_w2, pred_b2.reshape(1, -1))


def kernel(x, edge_index, cluster, valid_len, time_step_len,
           gl0_w1, gl0_b1, gl0_g, gl0_beta, gl0_w2, gl0_b2,
           gl1_w1, gl1_b1, gl1_g, gl1_beta, gl1_w2, gl1_b2,
           q_w, q_b, k_w, k_b, v_w, v_b,
           pred_w1, pred_b1, pred_g, pred_beta, pred_w2, pred_b2):
    src = edge_index[0]
    dst = edge_index[1]
    n = x.shape[0]

    h0 = _dense0(x, gl0_w1, gl0_b1, gl0_g, gl0_beta, gl0_w2, gl0_b2)
    agg0 = _segment_max(h0, src, dst)
    h1 = _dense1(h0, agg0, gl1_w1, gl1_b1, gl1_g, gl1_beta, gl1_w2, gl1_b2)
    agg1 = _segment_max(h1, src, dst)

    t = 50
    num_clusters = valid_len.shape[0] * t
    group = n // num_clusters
    pooled = _pool(h1, agg1, group)
    return _tail(pooled, valid_len, t, q_w, q_b, k_w, k_b, v_w, v_b,
                 pred_w1, pred_b1, pred_g, pred_beta, pred_w2, pred_b2)
